# baseline (speedup 1.0000x reference)
import jax
import jax.numpy as jnp
from jax import lax
from jax.experimental import pallas as pl
from jax.experimental.pallas import tpu as pltpu
from jax.experimental.pallas import tpu_sc as plsc

_NUM_ITEMS = 512
_BATCH = 16384
_DIM = 256
_OUT_ROWS = _NUM_ITEMS + _BATCH

_NC = 2
_NS = 16
_NW = _NC * _NS
_X_PER_W = _BATCH // _NW
_Q_PER_W = _NUM_ITEMS // _NW

_CHUNK = 64
_NCH = _X_PER_W // _CHUNK
_NBUF = 7


def _sc_copy(x_hbm, q_hbm, out_hbm, qv, b0, b1, b2, b3, b4, b5, b6, sq, *sems):
    wid = lax.axis_index("s") * _NC + lax.axis_index("c")
    qb = wid * _Q_PER_W
    xb = wid * _X_PER_W
    bufs = (b0, b1, b2, b3, b4, b5, b6)
    sin = sems[:_NBUF]
    sout = sems[_NBUF:]

    q_in = pltpu.make_async_copy(q_hbm.at[pl.ds(qb, _Q_PER_W)], qv, sq)
    q_in.start()

    gath = [None] * _NCH
    scat = [None] * _NCH
    gath[0] = pltpu.make_async_copy(
        x_hbm.at[pl.ds(xb, _CHUNK)], bufs[0], sin[0]
    )
    gath[0].start()

    q_in.wait()
    q_out = pltpu.make_async_copy(qv, out_hbm.at[pl.ds(qb, _Q_PER_W)], sq)
    q_out.start()

    for i in range(_NCH):
        if i + 1 < _NCH:
            nb = (i + 1) % _NBUF
            if i + 1 >= _NBUF:
                scat[i + 1 - _NBUF].wait()
            gath[i + 1] = pltpu.make_async_copy(
                x_hbm.at[pl.ds(xb + (i + 1) * _CHUNK, _CHUNK)], bufs[nb], sin[nb]
            )
            gath[i + 1].start()
        gath[i].wait()
        scat[i] = pltpu.make_async_copy(
            bufs[i % _NBUF],
            out_hbm.at[pl.ds(_NUM_ITEMS + xb + i * _CHUNK, _CHUNK)],
            sout[i % _NBUF],
        )
        scat[i].start()

    for i in range(max(0, _NCH - _NBUF + 1), _NCH):
        scat[i].wait()
    q_out.wait()


def kernel(x, q):
    run = pl.kernel(
        _sc_copy,
        out_type=jax.ShapeDtypeStruct((_OUT_ROWS, _DIM), jnp.float32),
        mesh=plsc.VectorSubcoreMesh(core_axis_name="c", subcore_axis_name="s"),
        scratch_types=(
            [pltpu.VMEM((_Q_PER_W, _DIM), jnp.float32)]
            + [pltpu.VMEM((_CHUNK, _DIM), jnp.float32) for _ in range(_NBUF)]
            + [pltpu.SemaphoreType.DMA for _ in range(2 * _NBUF + 1)]
        ),
    )
    return run(x, q)


# speedup vs baseline: 1.0251x; 1.0251x over previous
import jax
import jax.numpy as jnp
from jax import lax
from jax.experimental import pallas as pl
from jax.experimental.pallas import tpu as pltpu
from jax.experimental.pallas import tpu_sc as plsc

_NUM_ITEMS = 512
_BATCH = 16384
_DIM = 256
_OUT_ROWS = _NUM_ITEMS + _BATCH

_NC = 2
_NS = 16
_NW = _NC * _NS
_X_PER_W = _BATCH // _NW
_Q_PER_W = _NUM_ITEMS // _NW

_CHUNK = 128
_NCH = _X_PER_W // _CHUNK
_NBUF = 3


def _sc_copy(x_hbm, q_hbm, out_hbm, qv, b0, b1, b2, sq, *sems):
    wid = lax.axis_index("s") * _NC + lax.axis_index("c")
    qb = wid * _Q_PER_W
    xb = wid * _X_PER_W
    bufs = (b0, b1, b2)
    sin = sems[:_NBUF]
    sout = sems[_NBUF:]

    q_in = pltpu.make_async_copy(q_hbm.at[pl.ds(qb, _Q_PER_W)], qv, sq)
    q_in.start()

    gath = [None] * _NCH
    scat = [None] * _NCH
    gath[0] = pltpu.make_async_copy(
        x_hbm.at[pl.ds(xb, _CHUNK)], bufs[0], sin[0]
    )
    gath[0].start()

    q_in.wait()
    q_out = pltpu.make_async_copy(qv, out_hbm.at[pl.ds(qb, _Q_PER_W)], sq)
    q_out.start()

    for i in range(_NCH):
        if i + 1 < _NCH:
            nb = (i + 1) % _NBUF
            if i + 1 >= _NBUF:
                scat[i + 1 - _NBUF].wait()
            gath[i + 1] = pltpu.make_async_copy(
                x_hbm.at[pl.ds(xb + (i + 1) * _CHUNK, _CHUNK)], bufs[nb], sin[nb]
            )
            gath[i + 1].start()
        gath[i].wait()
        scat[i] = pltpu.make_async_copy(
            bufs[i % _NBUF],
            out_hbm.at[pl.ds(_NUM_ITEMS + xb + i * _CHUNK, _CHUNK)],
            sout[i % _NBUF],
        )
        scat[i].start()

    for i in range(max(0, _NCH - _NBUF + 1), _NCH):
        scat[i].wait()
    q_out.wait()


def kernel(x, q):
    run = pl.kernel(
        _sc_copy,
        out_type=jax.ShapeDtypeStruct((_OUT_ROWS, _DIM), jnp.float32),
        mesh=plsc.VectorSubcoreMesh(core_axis_name="c", subcore_axis_name="s"),
        scratch_types=(
            [pltpu.VMEM((_Q_PER_W, _DIM), jnp.float32)]
            + [pltpu.VMEM((_CHUNK, _DIM), jnp.float32) for _ in range(_NBUF)]
            + [pltpu.SemaphoreType.DMA for _ in range(2 * _NBUF + 1)]
        ),
    )
    return run(x, q)


# speedup vs baseline: 1.0256x; 1.0005x over previous
import jax
import jax.numpy as jnp
from jax import lax
from jax.experimental import pallas as pl
from jax.experimental.pallas import tpu as pltpu
from jax.experimental.pallas import tpu_sc as plsc

_NUM_ITEMS = 512
_BATCH = 16384
_DIM = 256
_OUT_ROWS = _NUM_ITEMS + _BATCH

_NC = 2
_NS = 16
_NW = _NC * _NS
_X_PER_W = _BATCH // _NW
_Q_PER_W = _NUM_ITEMS // _NW

_CHUNK = 128
_NCH = _X_PER_W // _CHUNK
_NBUF = 3


def _sc_copy(x_hbm, q_hbm, out_hbm, qv, b0, b1, b2, sq, *sems):
    wid = lax.axis_index("s") * _NC + lax.axis_index("c")
    qb = wid * _Q_PER_W
    xb = wid * _X_PER_W
    bufs = (b0, b1, b2)
    sin = sems[:_NBUF]
    sout = sems[_NBUF:]

    q_in = pltpu.make_async_copy(q_hbm.at[pl.ds(qb, _Q_PER_W)], qv, sq)
    q_in.start()

    gath = [None] * _NCH
    scat = [None] * _NCH
    gath[0] = pltpu.make_async_copy(
        x_hbm.at[pl.ds(xb, _CHUNK)], bufs[0], sin[0]
    )
    gath[0].start()

    q_in.wait()
    q_out = pltpu.make_async_copy(qv, out_hbm.at[pl.ds(qb, _Q_PER_W)], sq)
    q_out.start()

    for i in range(_NCH):
        if i + 1 < _NCH:
            nb = (i + 1) % _NBUF
            if i + 1 >= _NBUF:
                scat[i + 1 - _NBUF].wait()
            gath[i + 1] = pltpu.make_async_copy(
                x_hbm.at[pl.ds(xb + (i + 1) * _CHUNK, _CHUNK)], bufs[nb], sin[nb]
            )
            gath[i + 1].start()
        gath[i].wait()
        scat[i] = pltpu.make_async_copy(
            bufs[i % _NBUF],
            out_hbm.at[pl.ds(_NUM_ITEMS + xb + i * _CHUNK, _CHUNK)],
            sout[i % _NBUF],
        )
        scat[i].start()

    for i in range(max(0, _NCH - _NBUF), _NCH):
        scat[i].wait()
    q_out.wait()


def kernel(x, q):
    run = pl.kernel(
        _sc_copy,
        out_type=jax.ShapeDtypeStruct((_OUT_ROWS, _DIM), jnp.float32),
        mesh=plsc.VectorSubcoreMesh(core_axis_name="c", subcore_axis_name="s"),
        scratch_types=(
            [pltpu.VMEM((_Q_PER_W, _DIM), jnp.float32)]
            + [pltpu.VMEM((_CHUNK, _DIM), jnp.float32) for _ in range(_NBUF)]
            + [pltpu.SemaphoreType.DMA for _ in range(2 * _NBUF + 1)]
        ),
    )
    return run(x, q)
